# baseline (speedup 1.0000x reference)
import functools

import jax
import jax.numpy as jnp
from jax import lax
from jax.experimental import pallas as pl
from jax.experimental.pallas import tpu as pltpu
from jax.experimental.pallas import tpu_sc as plsc

_N = 4096
_E = 65536
_F = 16
_B = 8
_NUM_DATES = 1024
_NC = 2
_NS = 16
_NW = _NC * _NS
_EPW = _E // _NW
_CH = 128
_NCH = _EPW // _CH
_RPW = _N // _NS


def _sc_scatter_body(g_hbm, src_hbm, dst_hbm, zero_hbm, out_hbm,
                     src_v, dst_v, rows_v, acc_sh, sem, sem2):
    cid = lax.axis_index("c")
    sid = lax.axis_index("s")
    wid = cid * _NS + sid

    zc = pltpu.make_async_copy(zero_hbm.at[pl.ds(sid * _RPW, _RPW)],
                               acc_sh.at[pl.ds(sid * _RPW, _RPW)], sem2)
    zc.start()
    pltpu.sync_copy(src_hbm.at[wid], src_v)
    for j in range(_NCH):
        pltpu.async_copy(g_hbm.at[src_v.at[j]], rows_v.at[j], sem)
    pltpu.sync_copy(dst_hbm.at[wid], dst_v)
    zc.wait()
    plsc.subcore_barrier()
    for j in range(_NCH):
        pltpu.make_async_copy(g_hbm.at[src_v.at[j]], rows_v.at[j], sem).wait()
        pltpu.async_copy(rows_v.at[j], acc_sh.at[dst_v.at[j]], sem2, add=True)
    for j in range(_NCH):
        pltpu.make_async_copy(rows_v.at[j], acc_sh.at[dst_v.at[j]], sem2).wait()
    plsc.subcore_barrier()
    pltpu.sync_copy(acc_sh.at[pl.ds(sid * _RPW, _RPW)],
                    out_hbm.at[cid, pl.ds(sid * _RPW, _RPW)])


def _sc_counts_body(dst_hbm, ones_hbm, zero_hbm, out_hbm,
                    dst_v, ones_v, acc_sh, sem):
    cid = lax.axis_index("c")
    sid = lax.axis_index("s")
    wid = cid * _NS + sid
    zc = pltpu.make_async_copy(zero_hbm.at[pl.ds(sid * _RPW, _RPW)],
                               acc_sh.at[pl.ds(sid * _RPW, _RPW)], sem)
    zc.start()
    pltpu.sync_copy(dst_hbm.at[wid], dst_v)
    pltpu.sync_copy(ones_hbm.at[pl.ds(0, _CH)], ones_v)
    zc.wait()
    plsc.subcore_barrier()
    for j in range(_NCH):
        pltpu.async_copy(ones_v, acc_sh.at[dst_v.at[j]], sem, add=True)
    for j in range(_NCH):
        pltpu.make_async_copy(ones_v, acc_sh.at[dst_v.at[j]], sem).wait()
    plsc.subcore_barrier()
    pltpu.sync_copy(acc_sh.at[pl.ds(sid * _RPW, _RPW)],
                    out_hbm.at[cid, pl.ds(sid * _RPW, _RPW)])


@functools.cache
def _sc_counts_kernel():
    return pl.kernel(
        _sc_counts_body,
        out_type=jax.ShapeDtypeStruct((_NC, _N, _F), jnp.float32),
        mesh=plsc.VectorSubcoreMesh(core_axis_name="c", subcore_axis_name="s",
                                    num_cores=_NC, num_subcores=_NS),
        compiler_params=pltpu.CompilerParams(use_tc_tiling_on_sc=False),
        scratch_types=[
            pltpu.VMEM((_NCH, _CH), jnp.int32),
            pltpu.VMEM((_CH, _F), jnp.float32),
            pltpu.VMEM_SHARED((_N, _F), jnp.float32),
            pltpu.SemaphoreType.DMA,
        ],
    )


@functools.cache
def _sc_scatter_kernel():
    return pl.kernel(
        _sc_scatter_body,
        out_type=jax.ShapeDtypeStruct((_NC, _N, _F), jnp.float32),
        mesh=plsc.VectorSubcoreMesh(core_axis_name="c", subcore_axis_name="s",
                                    num_cores=_NC, num_subcores=_NS),
        compiler_params=pltpu.CompilerParams(use_tc_tiling_on_sc=False),
        scratch_types=[
            pltpu.VMEM((_NCH, _CH), jnp.int32),
            pltpu.VMEM((_NCH, _CH), jnp.int32),
            pltpu.VMEM((_NCH, _CH, _F), jnp.float32),
            pltpu.VMEM_SHARED((_N, _F), jnp.float32),
            pltpu.SemaphoreType.DMA,
            pltpu.SemaphoreType.DMA,
        ],
    )


def _sc_scatter(g, src, dst, zeros_nf):
    return _sc_scatter_kernel()(g, src, dst, zeros_nf)


def _mmT(a, b):
    return lax.dot_general(a, b, (((1,), (1,)), ((), ())),
                           preferred_element_type=jnp.float32)


def _mm(a, b):
    return lax.dot_general(a, b, (((1,), (0,)), ((), ())),
                           preferred_element_type=jnp.float32)


def _dinv(cp):
    return lax.rsqrt(cp[0] + cp[1] + 1.0)


def _unpack(ap):
    parts = [ap[:, b * 16:(b + 1) * 16] for b in range(8)]
    return jnp.stack(parts, axis=1).reshape(ap.shape[0] * 8, 16)


def _pack(x):
    x3 = x.reshape(x.shape[0] // 8, 8, 16)
    return jnp.concatenate([x3[:, b, :] for b in range(8)], axis=1)


def _k1_body(x_ref, w_ref, b_ref, cp_ref, o_ref):
    xw = _mmT(x_ref[...], w_ref[...])
    o_ref[...] = _pack(xw + b_ref[...]) * _dinv(cp_ref)


def _proj_in(x, w_in1, b_in1, cp_p):
    blk = 1024
    return pl.pallas_call(
        _k1_body,
        grid=(_N // blk,),
        in_specs=[
            pl.BlockSpec((blk, _N), lambda i: (i, 0)),
            pl.BlockSpec((_F, _N), lambda i: (0, 0)),
            pl.BlockSpec((1, _F), lambda i: (0, 0)),
            pl.BlockSpec((_NC, blk // 8, 128), lambda i: (0, i, 0)),
        ],
        out_specs=pl.BlockSpec((blk // 8, 128), lambda i: (i, 0)),
        out_shape=jax.ShapeDtypeStruct((_N // 8, 128), jnp.float32),
    )(x, w_in1, b_in1, cp_p)


def _g1_body(s1p, g1, cp, c1w, c1b, wt1, bt1, ep16, ets, ea, o_ref):
    dinv = _dinv(cp)
    agg1 = _unpack(dinv * (s1p[0] + s1p[1] + g1[...]))
    x2 = jnp.maximum(_mmT(agg1, c1w[...]) + c1b[...], 0.0)
    eav = ea[...]
    mx_e = jnp.max(eav)
    ids = (lax.broadcasted_iota(jnp.int32, eav.shape, 0) * eav.shape[1]
           + lax.broadcasted_iota(jnp.int32, eav.shape, 1))
    ts_idx = jnp.min(jnp.where(eav == mx_e, ids, _E))
    ts_idx = lax.rem(ts_idx, _NUM_DATES)
    ts_row = ets[pl.ds(ts_idx, 1), :]
    mx = jnp.max(x2, axis=1, keepdims=True)
    colid = lax.broadcasted_iota(jnp.int32, x2.shape, 1)
    pj = jnp.min(jnp.where(x2 == mx, colid, _F), axis=1, keepdims=True)
    oh = (pj == lax.broadcasted_iota(jnp.int32, x2.shape, 1)
          ).astype(jnp.float32)
    wt1v = wt1[...]
    t2 = _mmT(ep16[...], wt1v[:, 32:48])
    x3 = (_mmT(x2, wt1v[:, 0:16]) + _mm(oh, t2)
          + _mmT(ts_row, wt1v[:, 16:32]) + bt1[...])
    o_ref[...] = dinv * _pack(x3)


def _g2_body(s2p, g2, cp, c2w, c2b, o_ref):
    dinv = _dinv(cp)
    agg = _unpack(dinv * (s2p[0] + s2p[1] + g2[...]))
    x4 = jnp.maximum(_mmT(agg, c2w[...]) + c2b[...], 0.0)
    o_ref[...] = dinv * _pack(x4)


def _g3_body(s3p, g3, cp, c3w, c3b, w_in2, b_in2, a_in, a_out,
             wt2, bt2, wt3, bt3, f1w, f1b, f2w, f2b, batch2d, o_ref):
    dinv = _dinv(cp)
    aggv = _unpack(dinv * (s3p[0] + s3p[1] + g3[...]))
    m3 = lax.dot_general(c3w[...], w_in2[...], (((0,), (1,)), ((), ())),
                         preferred_element_type=jnp.float32)
    c3row = _mmT(c3b[...], w_in2[...])
    x5 = _mm(aggv, m3) + c3row + b_in2[...]
    li = jnp.maximum(lax.dot_general(a_in[...], x5, (((1,), (1,)), ((), ())),
                                     preferred_element_type=jnp.float32), 0.0)
    lo = jnp.maximum(lax.dot_general(a_out[...], x5, (((1,), (1,)), ((), ())),
                                     preferred_element_type=jnp.float32), 0.0)
    logits = jnp.concatenate([li, lo], axis=1)
    e = jnp.exp(logits - jnp.max(logits))
    attn = e / jnp.sum(e)
    ohb = (batch2d[...] == lax.broadcasted_iota(jnp.int32, (_N, _B), 1)
           ).astype(jnp.float32)
    cnt = lax.dot_general(ohb, jnp.ones((_N, 1), jnp.float32),
                          (((0,), (0,)), ((), ())),
                          preferred_element_type=jnp.float32)
    hsum = lax.dot_general(ohb, x5, (((0,), (0,)), ((), ())),
                           preferred_element_type=jnp.float32)
    hbar = hsum / jnp.maximum(cnt, 1.0)
    s = _mmT(hbar, wt2[...]) + bt2[...]
    s = jnp.where(cnt > 0.0, s, 0.0)
    y = _mmT(attn * s, wt3[...]) + bt3[...]
    h1 = jnp.maximum(_mmT(y, f1w[...]) + f1b[...], 0.0)
    o_ref[...] = _mmT(h1, f2w[...]) + f2b[...]


def kernel(x, edge_index, edge_attr, batch, emb_player, emb_ts, conv1_W,
           conv1_b, conv2_W, conv2_b, conv3_W, conv3_b, attn_in1, attn_out1,
           fc1_W, fc1_b, fc2_W, fc2_b, W_in1, b_in1, W_t1, b_t1, W_in2, b_in2,
           W_t2, b_t2, W_t3, b_t3):
    f32 = jnp.float32
    src = edge_index[0].reshape(_NW, _NCH, _CH)
    dst = edge_index[1].reshape(_NW, _NCH, _CH)
    zeros_nf = jnp.zeros((_N // 8, 128), f32).reshape(_N, _F)
    ones_nf = jnp.ones((_N // 8, 128), f32).reshape(_N, _F)

    pk = (_N // 8, 128)
    cp = _sc_counts_kernel()(dst, ones_nf, zeros_nf)
    cp_p = cp.reshape(_NC, *pk)
    g1_p = _proj_in(x, W_in1, b_in1.reshape(1, _F), cp_p)
    s1p = _sc_scatter(g1_p.reshape(_N, _F), src, dst, zeros_nf)
    g2_p = pl.pallas_call(
        _g1_body,
        out_shape=jax.ShapeDtypeStruct(pk, f32),
    )(s1p.reshape(_NC, *pk), g1_p, cp_p, conv1_W, conv1_b.reshape(1, _F),
      W_t1, b_t1.reshape(1, _F), emb_player[:_F], emb_ts,
      edge_attr.reshape(_E // 128, 128))
    s2p = _sc_scatter(g2_p.reshape(_N, _F), src, dst, zeros_nf)
    g3_p = pl.pallas_call(
        _g2_body,
        out_shape=jax.ShapeDtypeStruct(pk, f32),
    )(s2p.reshape(_NC, *pk), g2_p, cp_p, conv2_W, conv2_b.reshape(1, _F))
    s3p = _sc_scatter(g3_p.reshape(_N, _F), src, dst, zeros_nf)
    out = pl.pallas_call(
        _g3_body,
        out_shape=jax.ShapeDtypeStruct((_B, 3), f32),
    )(s3p.reshape(_NC, *pk), g3_p, cp_p, conv3_W, conv3_b.reshape(1, _N), W_in2,
      b_in2.reshape(1, _F), attn_in1, attn_out1, W_t2,
      b_t2.reshape(1, 2 * _N), W_t3, b_t3.reshape(1, _F), fc1_W,
      fc1_b.reshape(1, _F), fc2_W, fc2_b.reshape(1, 3), batch.reshape(_N, 1))
    return out


# speedup vs baseline: 1.0182x; 1.0182x over previous
import functools

import jax
import jax.numpy as jnp
from jax import lax
from jax.experimental import pallas as pl
from jax.experimental.pallas import tpu as pltpu
from jax.experimental.pallas import tpu_sc as plsc

_N = 4096
_E = 65536
_F = 16
_B = 8
_NUM_DATES = 1024
_NC = 2
_NS = 16
_NW = _NC * _NS
_EPW = _E // _NW
_CH = 128
_NCH = _EPW // _CH
_RPW = _N // _NS


def _sc_scatter_body(g_hbm, src_hbm, dst_hbm, zero_hbm, out_hbm,
                     src_v, dst_v, rows_v, acc_sh, sem, sem2):
    cid = lax.axis_index("c")
    sid = lax.axis_index("s")
    wid = cid * _NS + sid

    zc = pltpu.make_async_copy(zero_hbm.at[pl.ds(sid * _RPW, _RPW)],
                               acc_sh.at[pl.ds(sid * _RPW, _RPW)], sem2)
    zc.start()
    pltpu.sync_copy(src_hbm.at[wid], src_v)
    for j in range(_NCH):
        pltpu.async_copy(g_hbm.at[src_v.at[j]], rows_v.at[j], sem)
    pltpu.sync_copy(dst_hbm.at[wid], dst_v)
    zc.wait()
    plsc.subcore_barrier()
    for j in range(_NCH):
        pltpu.make_async_copy(g_hbm.at[src_v.at[j]], rows_v.at[j], sem).wait()
        pltpu.async_copy(rows_v.at[j], acc_sh.at[dst_v.at[j]], sem2, add=True)
    for j in range(_NCH):
        pltpu.make_async_copy(rows_v.at[j], acc_sh.at[dst_v.at[j]], sem2).wait()
    plsc.subcore_barrier()
    pltpu.sync_copy(acc_sh.at[pl.ds(sid * _RPW, _RPW)],
                    out_hbm.at[cid, pl.ds(sid * _RPW, _RPW)])


def _sc_counts_body(dst_hbm, ones_hbm, zero_hbm, out_hbm,
                    dst_v, ones_v, acc_sh, sem):
    cid = lax.axis_index("c")
    sid = lax.axis_index("s")
    wid = cid * _NS + sid
    zc = pltpu.make_async_copy(zero_hbm.at[pl.ds(sid * _RPW, _RPW)],
                               acc_sh.at[pl.ds(sid * _RPW, _RPW)], sem)
    zc.start()
    pltpu.sync_copy(dst_hbm.at[wid], dst_v)
    pltpu.sync_copy(ones_hbm.at[pl.ds(0, _CH)], ones_v)
    zc.wait()
    plsc.subcore_barrier()
    for j in range(_NCH):
        pltpu.async_copy(ones_v, acc_sh.at[dst_v.at[j]], sem, add=True)
    for j in range(_NCH):
        pltpu.make_async_copy(ones_v, acc_sh.at[dst_v.at[j]], sem).wait()
    plsc.subcore_barrier()
    pltpu.sync_copy(acc_sh.at[pl.ds(sid * _RPW, _RPW)],
                    out_hbm.at[cid, pl.ds(sid * _RPW, _RPW)])


@functools.cache
def _sc_counts_kernel():
    return pl.kernel(
        _sc_counts_body,
        out_type=jax.ShapeDtypeStruct((_NC, _N, _F), jnp.float32),
        mesh=plsc.VectorSubcoreMesh(core_axis_name="c", subcore_axis_name="s",
                                    num_cores=_NC, num_subcores=_NS),
        compiler_params=pltpu.CompilerParams(use_tc_tiling_on_sc=False),
        scratch_types=[
            pltpu.VMEM((_NCH, _CH), jnp.int32),
            pltpu.VMEM((_CH, _F), jnp.float32),
            pltpu.VMEM_SHARED((_N, _F), jnp.float32),
            pltpu.SemaphoreType.DMA,
        ],
    )


@functools.cache
def _sc_scatter_kernel():
    return pl.kernel(
        _sc_scatter_body,
        out_type=jax.ShapeDtypeStruct((_NC, _N, _F), jnp.float32),
        mesh=plsc.VectorSubcoreMesh(core_axis_name="c", subcore_axis_name="s",
                                    num_cores=_NC, num_subcores=_NS),
        compiler_params=pltpu.CompilerParams(use_tc_tiling_on_sc=False),
        scratch_types=[
            pltpu.VMEM((_NCH, _CH), jnp.int32),
            pltpu.VMEM((_NCH, _CH), jnp.int32),
            pltpu.VMEM((_NCH, _CH, _F), jnp.float32),
            pltpu.VMEM_SHARED((_N, _F), jnp.float32),
            pltpu.SemaphoreType.DMA,
            pltpu.SemaphoreType.DMA,
        ],
    )


def _sc_scatter(g, src, dst, zeros_nf):
    return _sc_scatter_kernel()(g, src, dst, zeros_nf)


def _mmT(a, b):
    return lax.dot_general(a, b, (((1,), (1,)), ((), ())),
                           preferred_element_type=jnp.float32)


def _mm(a, b):
    return lax.dot_general(a, b, (((1,), (0,)), ((), ())),
                           preferred_element_type=jnp.float32)


def _dinv(cp):
    return lax.rsqrt(cp[0] + cp[1] + 1.0)


def _unpack(ap):
    parts = [ap[:, b * 16:(b + 1) * 16] for b in range(8)]
    return jnp.stack(parts, axis=1).reshape(ap.shape[0] * 8, 16)


def _pack(x):
    x3 = x.reshape(x.shape[0] // 8, 8, 16)
    return jnp.concatenate([x3[:, b, :] for b in range(8)], axis=1)


def _k1_body(x_ref, w_ref, b_ref, cp_ref, o_ref):
    xw = _mmT(x_ref[...], w_ref[...])
    o_ref[...] = _pack(xw + b_ref[...]) * _dinv(cp_ref)


def _proj_in(x, w_in1, b_in1, cp_p):
    blk = 512
    return pl.pallas_call(
        _k1_body,
        grid=(_N // blk,),
        in_specs=[
            pl.BlockSpec((blk, _N), lambda i: (i, 0)),
            pl.BlockSpec((_F, _N), lambda i: (0, 0)),
            pl.BlockSpec((1, _F), lambda i: (0, 0)),
            pl.BlockSpec((_NC, blk // 8, 128), lambda i: (0, i, 0)),
        ],
        out_specs=pl.BlockSpec((blk // 8, 128), lambda i: (i, 0)),
        out_shape=jax.ShapeDtypeStruct((_N // 8, 128), jnp.float32),
    )(x, w_in1, b_in1, cp_p)


def _g1_body(s1p, g1, cp, c1w, c1b, wt1, bt1, ep16, ets, ea, o_ref):
    dinv = _dinv(cp)
    agg1 = _unpack(dinv * (s1p[0] + s1p[1] + g1[...]))
    x2 = jnp.maximum(_mmT(agg1, c1w[...]) + c1b[...], 0.0)
    eav = ea[...]
    mx_e = jnp.max(eav)
    ids = (lax.broadcasted_iota(jnp.int32, eav.shape, 0) * eav.shape[1]
           + lax.broadcasted_iota(jnp.int32, eav.shape, 1))
    ts_idx = jnp.min(jnp.where(eav == mx_e, ids, _E))
    ts_idx = lax.rem(ts_idx, _NUM_DATES)
    ts_row = ets[pl.ds(ts_idx, 1), :]
    mx = jnp.max(x2, axis=1, keepdims=True)
    colid = lax.broadcasted_iota(jnp.int32, x2.shape, 1)
    pj = jnp.min(jnp.where(x2 == mx, colid, _F), axis=1, keepdims=True)
    oh = (pj == lax.broadcasted_iota(jnp.int32, x2.shape, 1)
          ).astype(jnp.float32)
    wt1v = wt1[...]
    t2 = _mmT(ep16[...], wt1v[:, 32:48])
    x3 = (_mmT(x2, wt1v[:, 0:16]) + _mm(oh, t2)
          + _mmT(ts_row, wt1v[:, 16:32]) + bt1[...])
    o_ref[...] = dinv * _pack(x3)


def _g2_body(s2p, g2, cp, c2w, c2b, o_ref):
    dinv = _dinv(cp)
    agg = _unpack(dinv * (s2p[0] + s2p[1] + g2[...]))
    x4 = jnp.maximum(_mmT(agg, c2w[...]) + c2b[...], 0.0)
    o_ref[...] = dinv * _pack(x4)


def _g3_body(s3p, g3, cp, c3w, c3b, w_in2, b_in2, a_in, a_out,
             wt2, bt2, wt3, bt3, f1w, f1b, f2w, f2b, batch2d, o_ref):
    dinv = _dinv(cp)
    aggv = _unpack(dinv * (s3p[0] + s3p[1] + g3[...]))
    m3 = lax.dot_general(c3w[...], w_in2[...], (((0,), (1,)), ((), ())),
                         preferred_element_type=jnp.float32)
    c3row = _mmT(c3b[...], w_in2[...])
    x5 = _mm(aggv, m3) + c3row + b_in2[...]
    li = jnp.maximum(lax.dot_general(a_in[...], x5, (((1,), (1,)), ((), ())),
                                     preferred_element_type=jnp.float32), 0.0)
    lo = jnp.maximum(lax.dot_general(a_out[...], x5, (((1,), (1,)), ((), ())),
                                     preferred_element_type=jnp.float32), 0.0)
    logits = jnp.concatenate([li, lo], axis=1)
    e = jnp.exp(logits - jnp.max(logits))
    attn = e / jnp.sum(e)
    ohb = (batch2d[...] == lax.broadcasted_iota(jnp.int32, (_N, _B), 1)
           ).astype(jnp.float32)
    cnt = lax.dot_general(ohb, jnp.ones((_N, 1), jnp.float32),
                          (((0,), (0,)), ((), ())),
                          preferred_element_type=jnp.float32)
    hsum = lax.dot_general(ohb, x5, (((0,), (0,)), ((), ())),
                           preferred_element_type=jnp.float32)
    hbar = hsum / jnp.maximum(cnt, 1.0)
    s = _mmT(hbar, wt2[...]) + bt2[...]
    s = jnp.where(cnt > 0.0, s, 0.0)
    y = _mmT(attn * s, wt3[...]) + bt3[...]
    h1 = jnp.maximum(_mmT(y, f1w[...]) + f1b[...], 0.0)
    o_ref[...] = _mmT(h1, f2w[...]) + f2b[...]


def kernel(x, edge_index, edge_attr, batch, emb_player, emb_ts, conv1_W,
           conv1_b, conv2_W, conv2_b, conv3_W, conv3_b, attn_in1, attn_out1,
           fc1_W, fc1_b, fc2_W, fc2_b, W_in1, b_in1, W_t1, b_t1, W_in2, b_in2,
           W_t2, b_t2, W_t3, b_t3):
    f32 = jnp.float32
    src = edge_index[0].reshape(_NW, _NCH, _CH)
    dst = edge_index[1].reshape(_NW, _NCH, _CH)
    zeros_nf = jnp.zeros((_N // 8, 128), f32).reshape(_N, _F)
    ones_nf = jnp.ones((_N // 8, 128), f32).reshape(_N, _F)

    pk = (_N // 8, 128)
    cp = _sc_counts_kernel()(dst, ones_nf, zeros_nf)
    cp_p = cp.reshape(_NC, *pk)
    g1_p = _proj_in(x, W_in1, b_in1.reshape(1, _F), cp_p)
    s1p = _sc_scatter(g1_p.reshape(_N, _F), src, dst, zeros_nf)
    g2_p = pl.pallas_call(
        _g1_body,
        out_shape=jax.ShapeDtypeStruct(pk, f32),
    )(s1p.reshape(_NC, *pk), g1_p, cp_p, conv1_W, conv1_b.reshape(1, _F),
      W_t1, b_t1.reshape(1, _F), emb_player[:_F], emb_ts,
      edge_attr.reshape(_E // 128, 128))
    s2p = _sc_scatter(g2_p.reshape(_N, _F), src, dst, zeros_nf)
    g3_p = pl.pallas_call(
        _g2_body,
        out_shape=jax.ShapeDtypeStruct(pk, f32),
    )(s2p.reshape(_NC, *pk), g2_p, cp_p, conv2_W, conv2_b.reshape(1, _F))
    s3p = _sc_scatter(g3_p.reshape(_N, _F), src, dst, zeros_nf)
    out = pl.pallas_call(
        _g3_body,
        out_shape=jax.ShapeDtypeStruct((_B, 3), f32),
    )(s3p.reshape(_NC, *pk), g3_p, cp_p, conv3_W, conv3_b.reshape(1, _N), W_in2,
      b_in2.reshape(1, _F), attn_in1, attn_out1, W_t2,
      b_t2.reshape(1, 2 * _N), W_t3, b_t3.reshape(1, _F), fc1_W,
      fc1_b.reshape(1, _F), fc2_W, fc2_b.reshape(1, 3), batch.reshape(_N, 1))
    return out


# speedup vs baseline: 1.1149x; 1.0949x over previous
import functools

import jax
import jax.numpy as jnp
from jax import lax
from jax.experimental import pallas as pl
from jax.experimental.pallas import tpu as pltpu
from jax.experimental.pallas import tpu_sc as plsc

_N = 4096
_E = 65536
_F = 16
_B = 8
_NUM_DATES = 1024
_NC = 2
_NS = 16
_NW = _NC * _NS
_EPW = _E // _NW
_CH = 128
_NCH = _EPW // _CH
_RPW = _N // _NS


def _sc_scatter_body(g_hbm, src_hbm, dst_hbm, zero_hbm, out_hbm,
                     src_v, dst_v, rows_v, acc_sh, sem, sem2):
    cid = lax.axis_index("c")
    sid = lax.axis_index("s")
    wid = cid * _NS + sid

    zc = pltpu.make_async_copy(zero_hbm.at[pl.ds(sid * _RPW, _RPW)],
                               acc_sh.at[pl.ds(sid * _RPW, _RPW)], sem2)
    zc.start()
    pltpu.sync_copy(src_hbm.at[wid], src_v)
    for j in range(_NCH):
        pltpu.async_copy(g_hbm.at[src_v.at[j]], rows_v.at[j], sem)
    pltpu.sync_copy(dst_hbm.at[wid], dst_v)
    zc.wait()
    plsc.subcore_barrier()
    for j in range(_NCH):
        pltpu.make_async_copy(g_hbm.at[src_v.at[j]], rows_v.at[j], sem).wait()
        pltpu.async_copy(rows_v.at[j], acc_sh.at[dst_v.at[j]], sem2, add=True)
    for j in range(_NCH):
        pltpu.make_async_copy(rows_v.at[j], acc_sh.at[dst_v.at[j]], sem2).wait()
    plsc.subcore_barrier()
    pltpu.sync_copy(acc_sh.at[pl.ds(sid * _RPW, _RPW)],
                    out_hbm.at[cid, pl.ds(sid * _RPW, _RPW)])


def _sc_counts_body(dst_hbm, ones_hbm, zero_hbm, out_hbm,
                    dst_v, ones_v, acc_sh, sem):
    cid = lax.axis_index("c")
    sid = lax.axis_index("s")
    wid = cid * _NS + sid
    zc = pltpu.make_async_copy(zero_hbm.at[pl.ds(sid * _RPW, _RPW)],
                               acc_sh.at[pl.ds(sid * _RPW, _RPW)], sem)
    zc.start()
    pltpu.sync_copy(dst_hbm.at[wid], dst_v)
    pltpu.sync_copy(ones_hbm.at[pl.ds(0, _CH)], ones_v)
    zc.wait()
    plsc.subcore_barrier()
    for j in range(_NCH):
        pltpu.async_copy(ones_v, acc_sh.at[dst_v.at[j]], sem, add=True)
    for j in range(_NCH):
        pltpu.make_async_copy(ones_v, acc_sh.at[dst_v.at[j]], sem).wait()
    plsc.subcore_barrier()
    pltpu.sync_copy(acc_sh.at[pl.ds(sid * _RPW, _RPW)],
                    out_hbm.at[cid, pl.ds(sid * _RPW, _RPW)])


@functools.cache
def _sc_counts_kernel():
    return pl.kernel(
        _sc_counts_body,
        out_type=jax.ShapeDtypeStruct((_NC, _N, _F), jnp.float32),
        mesh=plsc.VectorSubcoreMesh(core_axis_name="c", subcore_axis_name="s",
                                    num_cores=_NC, num_subcores=_NS),
        compiler_params=pltpu.CompilerParams(use_tc_tiling_on_sc=False),
        scratch_types=[
            pltpu.VMEM((_NCH, _CH), jnp.int32),
            pltpu.VMEM((_CH, _F), jnp.float32),
            pltpu.VMEM_SHARED((_N, _F), jnp.float32),
            pltpu.SemaphoreType.DMA,
        ],
    )


@functools.cache
def _sc_scatter_kernel():
    return pl.kernel(
        _sc_scatter_body,
        out_type=jax.ShapeDtypeStruct((_NC, _N, _F), jnp.float32),
        mesh=plsc.VectorSubcoreMesh(core_axis_name="c", subcore_axis_name="s",
                                    num_cores=_NC, num_subcores=_NS),
        compiler_params=pltpu.CompilerParams(use_tc_tiling_on_sc=False),
        scratch_types=[
            pltpu.VMEM((_NCH, _CH), jnp.int32),
            pltpu.VMEM((_NCH, _CH), jnp.int32),
            pltpu.VMEM((_NCH, _CH, _F), jnp.float32),
            pltpu.VMEM_SHARED((_N, _F), jnp.float32),
            pltpu.SemaphoreType.DMA,
            pltpu.SemaphoreType.DMA,
        ],
    )


def _sc_scatter(g, src, dst, zeros_nf):
    return _sc_scatter_kernel()(g, src, dst, zeros_nf)


def _mmT(a, b):
    return lax.dot_general(a, b, (((1,), (1,)), ((), ())),
                           preferred_element_type=jnp.float32)


def _mm(a, b):
    return lax.dot_general(a, b, (((1,), (0,)), ((), ())),
                           preferred_element_type=jnp.float32)


def _dinv(cp):
    return lax.rsqrt(cp[0] + cp[1] + 1.0)


def _unpack(ap):
    parts = [ap[:, b * 16:(b + 1) * 16] for b in range(8)]
    return jnp.stack(parts, axis=1).reshape(ap.shape[0] * 8, 16)


def _pack(x):
    x3 = x.reshape(x.shape[0] // 8, 8, 16)
    return jnp.concatenate([x3[:, b, :] for b in range(8)], axis=1)


def _unpack_perm(ap):
    return jnp.concatenate([ap[:, b * 16:(b + 1) * 16] for b in range(8)],
                           axis=0)


def _pack_perm(x):
    r = x.shape[0] // 8
    return jnp.concatenate([x[b * r:(b + 1) * r, :] for b in range(8)],
                           axis=1)


def _kron8(w):
    rep = jnp.concatenate([w] * 8, axis=0)
    rep = jnp.concatenate([rep] * 8, axis=1)
    r = lax.broadcasted_iota(jnp.int32, (128, 128), 0) // 16
    c = lax.broadcasted_iota(jnp.int32, (128, 128), 1) // 16
    return jnp.where(r == c, rep, 0.0)


def _tile8(v):
    return jnp.concatenate([v] * 8, axis=1)


def _k1_body(x_ref, w_ref, b_ref, cp_ref, o_ref):
    xw = _mmT(x_ref[...], w_ref[...])
    o_ref[...] = _pack(xw + b_ref[...]) * _dinv(cp_ref)


def _proj_in(x, w_in1, b_in1, cp_p):
    blk = 512
    return pl.pallas_call(
        _k1_body,
        grid=(_N // blk,),
        in_specs=[
            pl.BlockSpec((blk, _N), lambda i: (i, 0)),
            pl.BlockSpec((_F, _N), lambda i: (0, 0)),
            pl.BlockSpec((1, _F), lambda i: (0, 0)),
            pl.BlockSpec((_NC, blk // 8, 128), lambda i: (0, i, 0)),
        ],
        out_specs=pl.BlockSpec((blk // 8, 128), lambda i: (i, 0)),
        out_shape=jax.ShapeDtypeStruct((_N // 8, 128), jnp.float32),
    )(x, w_in1, b_in1, cp_p)


def _g1_body(s1p, g1, cp, c1w, c1b, wt1, bt1, ep16, ets, ea, o_ref):
    dinv = _dinv(cp)
    agg_p = dinv * (s1p[0] + s1p[1] + g1[...])
    x2_p = jnp.maximum(_mmT(agg_p, _kron8(c1w[...])) + _tile8(c1b[...]), 0.0)
    eav = ea[...]
    mx_e = jnp.max(eav)
    ids = (lax.broadcasted_iota(jnp.int32, eav.shape, 0) * eav.shape[1]
           + lax.broadcasted_iota(jnp.int32, eav.shape, 1))
    ts_idx = jnp.min(jnp.where(eav == mx_e, ids, _E))
    ts_idx = lax.rem(ts_idx, _NUM_DATES)
    ts_row = ets[pl.ds(ts_idx, 1), :]
    x2 = _unpack_perm(x2_p)
    mx = jnp.max(x2, axis=1, keepdims=True)
    colid = lax.broadcasted_iota(jnp.int32, x2.shape, 1)
    pj = jnp.min(jnp.where(x2 == mx, colid, _F), axis=1, keepdims=True)
    oh = (pj == lax.broadcasted_iota(jnp.int32, x2.shape, 1)
          ).astype(jnp.float32)
    oh_p = _pack_perm(oh)
    wt1v = wt1[...]
    t2 = _mmT(ep16[...], wt1v[:, 32:48])
    x3_p = (_mmT(x2_p, _kron8(wt1v[:, 0:16])) + _mm(oh_p, _kron8(t2))
            + _tile8(_mmT(ts_row, wt1v[:, 16:32]) + bt1[...]))
    o_ref[...] = dinv * x3_p


def _g2_body(s2p, g2, cp, c2w, c2b, o_ref):
    dinv = _dinv(cp)
    agg_p = dinv * (s2p[0] + s2p[1] + g2[...])
    x4_p = jnp.maximum(_mmT(agg_p, _kron8(c2w[...])) + _tile8(c2b[...]), 0.0)
    o_ref[...] = dinv * x4_p


def _g3_body(s3p, g3, cp, c3w, c3b, w_in2, b_in2, a_in, a_out,
             wt2, bt2, wt3, bt3, f1w, f1b, f2w, f2b, batch2d, o_ref):
    dinv = _dinv(cp)
    aggv = _unpack(dinv * (s3p[0] + s3p[1] + g3[...]))
    m3 = lax.dot_general(c3w[...], w_in2[...], (((0,), (1,)), ((), ())),
                         preferred_element_type=jnp.float32)
    c3row = _mmT(c3b[...], w_in2[...])
    x5 = _mm(aggv, m3) + c3row + b_in2[...]
    li = jnp.maximum(lax.dot_general(a_in[...], x5, (((1,), (1,)), ((), ())),
                                     preferred_element_type=jnp.float32), 0.0)
    lo = jnp.maximum(lax.dot_general(a_out[...], x5, (((1,), (1,)), ((), ())),
                                     preferred_element_type=jnp.float32), 0.0)
    logits = jnp.concatenate([li, lo], axis=1)
    e = jnp.exp(logits - jnp.max(logits))
    attn = e / jnp.sum(e)
    ohb = (batch2d[...] == lax.broadcasted_iota(jnp.int32, (_N, _B), 1)
           ).astype(jnp.float32)
    cnt = lax.dot_general(ohb, jnp.ones((_N, 1), jnp.float32),
                          (((0,), (0,)), ((), ())),
                          preferred_element_type=jnp.float32)
    hsum = lax.dot_general(ohb, x5, (((0,), (0,)), ((), ())),
                           preferred_element_type=jnp.float32)
    hbar = hsum / jnp.maximum(cnt, 1.0)
    s = _mmT(hbar, wt2[...]) + bt2[...]
    s = jnp.where(cnt > 0.0, s, 0.0)
    y = _mmT(attn * s, wt3[...]) + bt3[...]
    h1 = jnp.maximum(_mmT(y, f1w[...]) + f1b[...], 0.0)
    o_ref[...] = _mmT(h1, f2w[...]) + f2b[...]


def kernel(x, edge_index, edge_attr, batch, emb_player, emb_ts, conv1_W,
           conv1_b, conv2_W, conv2_b, conv3_W, conv3_b, attn_in1, attn_out1,
           fc1_W, fc1_b, fc2_W, fc2_b, W_in1, b_in1, W_t1, b_t1, W_in2, b_in2,
           W_t2, b_t2, W_t3, b_t3):
    f32 = jnp.float32
    src = edge_index[0].reshape(_NW, _NCH, _CH)
    dst = edge_index[1].reshape(_NW, _NCH, _CH)
    zeros_nf = jnp.zeros((_N // 8, 128), f32).reshape(_N, _F)
    ones_nf = jnp.ones((_N // 8, 128), f32).reshape(_N, _F)

    pk = (_N // 8, 128)
    cp = _sc_counts_kernel()(dst, ones_nf, zeros_nf)
    cp_p = cp.reshape(_NC, *pk)
    g1_p = _proj_in(x, W_in1, b_in1.reshape(1, _F), cp_p)
    s1p = _sc_scatter(g1_p.reshape(_N, _F), src, dst, zeros_nf)
    g2_p = pl.pallas_call(
        _g1_body,
        out_shape=jax.ShapeDtypeStruct(pk, f32),
    )(s1p.reshape(_NC, *pk), g1_p, cp_p, conv1_W, conv1_b.reshape(1, _F),
      W_t1, b_t1.reshape(1, _F), emb_player[:_F], emb_ts,
      edge_attr.reshape(_E // 128, 128))
    s2p = _sc_scatter(g2_p.reshape(_N, _F), src, dst, zeros_nf)
    g3_p = pl.pallas_call(
        _g2_body,
        out_shape=jax.ShapeDtypeStruct(pk, f32),
    )(s2p.reshape(_NC, *pk), g2_p, cp_p, conv2_W, conv2_b.reshape(1, _F))
    s3p = _sc_scatter(g3_p.reshape(_N, _F), src, dst, zeros_nf)
    out = pl.pallas_call(
        _g3_body,
        out_shape=jax.ShapeDtypeStruct((_B, 3), f32),
    )(s3p.reshape(_NC, *pk), g3_p, cp_p, conv3_W, conv3_b.reshape(1, _N), W_in2,
      b_in2.reshape(1, _F), attn_in1, attn_out1, W_t2,
      b_t2.reshape(1, 2 * _N), W_t3, b_t3.reshape(1, _F), fc1_W,
      fc1_b.reshape(1, _F), fc2_W, fc2_b.reshape(1, 3), batch.reshape(_N, 1))
    return out


# speedup vs baseline: 1.1287x; 1.0124x over previous
import functools

import jax
import jax.numpy as jnp
from jax import lax
from jax.experimental import pallas as pl
from jax.experimental.pallas import tpu as pltpu
from jax.experimental.pallas import tpu_sc as plsc

_N = 4096
_E = 65536
_F = 16
_B = 8
_NUM_DATES = 1024
_NC = 2
_NS = 16
_NW = _NC * _NS
_EPW = _E // _NW
_CH = 128
_NCH = _EPW // _CH
_RPW = _N // _NS


def _sc_scatter_body(g_hbm, src_hbm, dst_hbm, zero_hbm, out_hbm,
                     src_v, dst_v, rows_v, acc_sh, sem, sem2):
    cid = lax.axis_index("c")
    sid = lax.axis_index("s")
    wid = cid * _NS + sid

    zc = pltpu.make_async_copy(zero_hbm.at[pl.ds(sid * _RPW, _RPW)],
                               acc_sh.at[pl.ds(sid * _RPW, _RPW)], sem2)
    zc.start()
    pltpu.sync_copy(src_hbm.at[wid], src_v)
    for j in range(_NCH):
        pltpu.async_copy(g_hbm.at[src_v.at[j]], rows_v.at[j], sem)
    pltpu.sync_copy(dst_hbm.at[wid], dst_v)
    zc.wait()
    plsc.subcore_barrier()
    for j in range(_NCH):
        pltpu.make_async_copy(g_hbm.at[src_v.at[j]], rows_v.at[j], sem).wait()
        pltpu.async_copy(rows_v.at[j], acc_sh.at[dst_v.at[j]], sem2, add=True)
    for j in range(_NCH):
        pltpu.make_async_copy(rows_v.at[j], acc_sh.at[dst_v.at[j]], sem2).wait()
    plsc.subcore_barrier()
    pltpu.sync_copy(acc_sh.at[pl.ds(sid * _RPW, _RPW)],
                    out_hbm.at[cid, pl.ds(sid * _RPW, _RPW)])


def _sc_counts_body(dst_hbm, ones_hbm, zero_hbm, out_hbm,
                    dst_v, ones_v, acc_sh, sem):
    cid = lax.axis_index("c")
    sid = lax.axis_index("s")
    wid = cid * _NS + sid
    zc = pltpu.make_async_copy(zero_hbm.at[pl.ds(sid * _RPW, _RPW)],
                               acc_sh.at[pl.ds(sid * _RPW, _RPW)], sem)
    zc.start()
    pltpu.sync_copy(dst_hbm.at[wid], dst_v)
    pltpu.sync_copy(ones_hbm.at[pl.ds(0, _CH)], ones_v)
    zc.wait()
    plsc.subcore_barrier()
    for j in range(_NCH):
        pltpu.async_copy(ones_v, acc_sh.at[dst_v.at[j]], sem, add=True)
    for j in range(_NCH):
        pltpu.make_async_copy(ones_v, acc_sh.at[dst_v.at[j]], sem).wait()
    plsc.subcore_barrier()
    pltpu.sync_copy(acc_sh.at[pl.ds(sid * _RPW, _RPW)],
                    out_hbm.at[cid, pl.ds(sid * _RPW, _RPW)])


@functools.cache
def _sc_counts_kernel():
    return pl.kernel(
        _sc_counts_body,
        out_type=jax.ShapeDtypeStruct((_NC, _N, _F), jnp.float32),
        mesh=plsc.VectorSubcoreMesh(core_axis_name="c", subcore_axis_name="s",
                                    num_cores=_NC, num_subcores=_NS),
        compiler_params=pltpu.CompilerParams(use_tc_tiling_on_sc=False),
        scratch_types=[
            pltpu.VMEM((_NCH, _CH), jnp.int32),
            pltpu.VMEM((_CH, _F), jnp.float32),
            pltpu.VMEM_SHARED((_N, _F), jnp.float32),
            pltpu.SemaphoreType.DMA,
        ],
    )


@functools.cache
def _sc_scatter_kernel():
    return pl.kernel(
        _sc_scatter_body,
        out_type=jax.ShapeDtypeStruct((_NC, _N, _F), jnp.float32),
        mesh=plsc.VectorSubcoreMesh(core_axis_name="c", subcore_axis_name="s",
                                    num_cores=_NC, num_subcores=_NS),
        compiler_params=pltpu.CompilerParams(use_tc_tiling_on_sc=False),
        scratch_types=[
            pltpu.VMEM((_NCH, _CH), jnp.int32),
            pltpu.VMEM((_NCH, _CH), jnp.int32),
            pltpu.VMEM((_NCH, _CH, _F), jnp.float32),
            pltpu.VMEM_SHARED((_N, _F), jnp.float32),
            pltpu.SemaphoreType.DMA,
            pltpu.SemaphoreType.DMA,
        ],
    )


def _sc_scatter(g, src, dst, zeros_nf):
    return _sc_scatter_kernel()(g, src, dst, zeros_nf)


def _mmT(a, b):
    return lax.dot_general(a, b, (((1,), (1,)), ((), ())),
                           preferred_element_type=jnp.float32)


def _mm(a, b):
    return lax.dot_general(a, b, (((1,), (0,)), ((), ())),
                           preferred_element_type=jnp.float32)


def _dinv(cp):
    return 1.0 / jnp.sqrt(cp[0] + cp[1] + 1.0)


def _unpack(ap):
    parts = [ap[:, b * 16:(b + 1) * 16] for b in range(8)]
    return jnp.stack(parts, axis=1).reshape(ap.shape[0] * 8, 16)


def _pack(x):
    x3 = x.reshape(x.shape[0] // 8, 8, 16)
    return jnp.concatenate([x3[:, b, :] for b in range(8)], axis=1)


def _unpack_perm(ap):
    return jnp.concatenate([ap[:, b * 16:(b + 1) * 16] for b in range(8)],
                           axis=0)


def _pack_perm(x):
    r = x.shape[0] // 8
    return jnp.concatenate([x[b * r:(b + 1) * r, :] for b in range(8)],
                           axis=1)


def _kron8(w):
    rep = jnp.concatenate([w] * 8, axis=0)
    rep = jnp.concatenate([rep] * 8, axis=1)
    r = lax.broadcasted_iota(jnp.int32, (128, 128), 0) // 16
    c = lax.broadcasted_iota(jnp.int32, (128, 128), 1) // 16
    return jnp.where(r == c, rep, 0.0)


def _tile8(v):
    return jnp.concatenate([v] * 8, axis=1)


def _k1_body(x_ref, w_ref, b_ref, c1w_ref, cp_ref, o_ref):
    x1 = _mmT(x_ref[...], w_ref[...]) + b_ref[...]
    o_ref[...] = _pack(_mmT(x1, c1w_ref[...])) * _dinv(cp_ref)


def _proj_in(x, w_in1, b_in1, c1w, cp_p):
    blk = 512
    return pl.pallas_call(
        _k1_body,
        grid=(_N // blk,),
        in_specs=[
            pl.BlockSpec((blk, _N), lambda i: (i, 0)),
            pl.BlockSpec((_F, _N), lambda i: (0, 0)),
            pl.BlockSpec((1, _F), lambda i: (0, 0)),
            pl.BlockSpec((_F, _F), lambda i: (0, 0)),
            pl.BlockSpec((_NC, blk // 8, 128), lambda i: (0, i, 0)),
        ],
        out_specs=pl.BlockSpec((blk // 8, 128), lambda i: (i, 0)),
        out_shape=jax.ShapeDtypeStruct((_N // 8, 128), jnp.float32),
    )(x, w_in1, b_in1, c1w, cp_p)


def _g1_body(s1p, g1, cp, c1b, wt1, bt1, ep16, ets, ea, c2w, o_ref):
    dinv = _dinv(cp)
    x2_p = jnp.maximum(dinv * (s1p[0] + s1p[1] + g1[...]) + _tile8(c1b[...]),
                       0.0)
    eav = ea[...]
    mx_e = jnp.max(eav)
    ids = (lax.broadcasted_iota(jnp.int32, eav.shape, 0) * eav.shape[1]
           + lax.broadcasted_iota(jnp.int32, eav.shape, 1))
    ts_idx = jnp.min(jnp.where(eav == mx_e, ids, _E))
    ts_idx = lax.rem(ts_idx, _NUM_DATES)
    ts_row = ets[pl.ds(ts_idx, 1), :]
    x2 = _unpack_perm(x2_p)
    mx = jnp.max(x2, axis=1, keepdims=True)
    colid = lax.broadcasted_iota(jnp.int32, x2.shape, 1)
    pj = jnp.min(jnp.where(x2 == mx, colid, _F), axis=1, keepdims=True)
    oh = (pj == lax.broadcasted_iota(jnp.int32, x2.shape, 1)
          ).astype(jnp.float32)
    oh_p = _pack_perm(oh)
    wt1v = wt1[...]
    t2 = _mmT(ep16[...], wt1v[:, 32:48])
    x3_p = (_mmT(x2_p, _kron8(wt1v[:, 0:16])) + _mm(oh_p, _kron8(t2))
            + _tile8(_mmT(ts_row, wt1v[:, 16:32]) + bt1[...]))
    o_ref[...] = dinv * _mmT(x3_p, _kron8(c2w[...]))


def _g2_body(s2p, g2, cp, c2b, o_ref):
    dinv = _dinv(cp)
    x4_p = jnp.maximum(dinv * (s2p[0] + s2p[1] + g2[...]) + _tile8(c2b[...]),
                       0.0)
    o_ref[...] = dinv * x4_p


def _g3_body(s3p, g3, cp, c3w, c3b, w_in2, b_in2, a_in,
             wt2, bt2, wt3, bt3, f1w, f1b, f2w, f2b, batch2d, o_ref):
    dinv = _dinv(cp)
    aggv = _unpack(dinv * (s3p[0] + s3p[1] + g3[...]))
    m3 = lax.dot_general(c3w[...], w_in2[...], (((0,), (1,)), ((), ())),
                         preferred_element_type=jnp.float32)
    c3row = _mmT(c3b[...], w_in2[...])
    x5 = _mm(aggv, m3) + c3row + b_in2[...]
    lg = jnp.maximum(lax.dot_general(a_in[...], x5, (((1,), (1,)), ((), ())),
                                     preferred_element_type=jnp.float32), 0.0)
    logits = jnp.concatenate([lg[0:1, :], lg[1:2, :]], axis=1)
    e = jnp.exp(logits - jnp.max(logits))
    attn = e / jnp.sum(e)
    ohb = (batch2d[...] == lax.broadcasted_iota(jnp.int32, (_N, _B), 1)
           ).astype(jnp.float32)
    cnt = lax.dot_general(ohb, jnp.ones((_N, 1), jnp.float32),
                          (((0,), (0,)), ((), ())),
                          preferred_element_type=jnp.float32)
    hsum = lax.dot_general(ohb, x5, (((0,), (0,)), ((), ())),
                           preferred_element_type=jnp.float32)
    hbar = hsum / jnp.maximum(cnt, 1.0)
    s = _mmT(hbar, wt2[...]) + bt2[...]
    s = jnp.where(cnt > 0.0, s, 0.0)
    y = _mmT(attn * s, wt3[...]) + bt3[...]
    h1 = jnp.maximum(_mmT(y, f1w[...]) + f1b[...], 0.0)
    o_ref[...] = _mmT(h1, f2w[...]) + f2b[...]


def kernel(x, edge_index, edge_attr, batch, emb_player, emb_ts, conv1_W,
           conv1_b, conv2_W, conv2_b, conv3_W, conv3_b, attn_in1, attn_out1,
           fc1_W, fc1_b, fc2_W, fc2_b, W_in1, b_in1, W_t1, b_t1, W_in2, b_in2,
           W_t2, b_t2, W_t3, b_t3):
    f32 = jnp.float32
    src = edge_index[0].reshape(_NW, _NCH, _CH)
    dst = edge_index[1].reshape(_NW, _NCH, _CH)
    zeros_nf = jnp.zeros((_N // 8, 128), f32).reshape(_N, _F)
    ones_nf = jnp.ones((_N // 8, 128), f32).reshape(_N, _F)

    pk = (_N // 8, 128)
    cp = _sc_counts_kernel()(dst, ones_nf, zeros_nf)
    cp_p = cp.reshape(_NC, *pk)
    g1_p = _proj_in(x, W_in1, b_in1.reshape(1, _F), conv1_W, cp_p)
    s1p = _sc_scatter(g1_p.reshape(_N, _F), src, dst, zeros_nf)
    g2_p = pl.pallas_call(
        _g1_body,
        out_shape=jax.ShapeDtypeStruct(pk, f32),
    )(s1p.reshape(_NC, *pk), g1_p, cp_p, conv1_b.reshape(1, _F),
      W_t1, b_t1.reshape(1, _F), emb_player[:_F], emb_ts,
      edge_attr.reshape(_E // 128, 128), conv2_W)
    s2p = _sc_scatter(g2_p.reshape(_N, _F), src, dst, zeros_nf)
    g3_p = pl.pallas_call(
        _g2_body,
        out_shape=jax.ShapeDtypeStruct(pk, f32),
    )(s2p.reshape(_NC, *pk), g2_p, cp_p, conv2_b.reshape(1, _F))
    s3p = _sc_scatter(g3_p.reshape(_N, _F), src, dst, zeros_nf)
    out = pl.pallas_call(
        _g3_body,
        out_shape=jax.ShapeDtypeStruct((_B, 3), f32),
    )(s3p.reshape(_NC, *pk), g3_p, cp_p, conv3_W, conv3_b.reshape(1, _N), W_in2,
      b_in2.reshape(1, _F), jnp.concatenate([attn_in1, attn_out1], axis=0), W_t2,
      b_t2.reshape(1, 2 * _N), W_t3, b_t3.reshape(1, _F), fc1_W,
      fc1_b.reshape(1, _F), fc2_W, fc2_b.reshape(1, 3), batch.reshape(_N, 1))
    return out
